# Initial kernel scaffold; baseline (speedup 1.0000x reference)
import jax
import jax.numpy as jnp
from jax.experimental import pallas as pl


def kernel(x, edge_index, W_l1, W_r1, b1, g1, be1, W_l2, W_r2, b2, g2, be2, W_l3, W_r3, b3):
    raise NotImplementedError("write your pallas kernel here")


# speedup vs baseline: 3.1177x; 3.1177x over previous
import functools

import jax
import jax.numpy as jnp
from jax import lax
from jax.experimental import pallas as pl
from jax.experimental.pallas import tpu as pltpu, tpu_sc as plsc

N = 10000
E = 320000
DIN = 128
DH = 128
DOUT = 64
EPS = 1e-5

NC = 2
NS = 16
NW = NC * NS
B = 128
T = 80
E_PAD = NW * T * B
N_ROWS = 10240
RPT = N_ROWS // NS

BN = 400
GRID = N // BN


def _make_sc_seg_sum(D):
    mesh = plsc.VectorSubcoreMesh(core_axis_name="c", subcore_axis_name="s")

    @functools.partial(
        pl.kernel,
        out_type=jax.ShapeDtypeStruct((NC, N_ROWS, D), jnp.float32),
        mesh=mesh,
        compiler_params=pltpu.CompilerParams(use_tc_tiling_on_sc=False),
        scratch_types=[
            pltpu.VMEM((2, B), jnp.int32),
            pltpu.VMEM((2, B), jnp.int32),
            pltpu.VMEM((2, B, D), jnp.float32),
            pltpu.VMEM_SHARED((N_ROWS, D), jnp.float32),
            pltpu.SemaphoreType.DMA,
        ],
    )
    def seg_sum(table, src_idx, dst_idx, zeros, out, src_v, dst_v, rows, acc, sem):
        c = lax.axis_index("c")
        s = lax.axis_index("s")
        w = c * NS + s

        pltpu.sync_copy(zeros, acc.at[pl.ds(s * RPT, RPT)])
        plsc.subcore_barrier()

        def body(t, carry):
            pltpu.sync_copy(src_idx.at[w, t], src_v.at[0])
            pltpu.sync_copy(dst_idx.at[w, t], dst_v.at[0])
            pltpu.async_copy(table.at[src_v.at[0]], rows.at[0], sem).wait()
            pltpu.sync_copy(rows.at[0], acc.at[dst_v.at[0]], add=True)
            return carry

        lax.fori_loop(0, T, body, 0)
        plsc.subcore_barrier()

        pltpu.sync_copy(acc.at[pl.ds(s * RPT, RPT)],
                        out.at[c, pl.ds(s * RPT, RPT)])

    return seg_sum


_sc_seg_sum_144 = _make_sc_seg_sum(DH + 16)
_sc_seg_sum_128 = _make_sc_seg_sum(DH)
_sc_seg_sum_64 = _make_sc_seg_sum(DOUT)


def _row_spec(d):
    return pl.BlockSpec((BN, d), lambda i: (i, 0))


def _full_spec(r, d):
    return pl.BlockSpec((r, d), lambda i: (0, 0))


def _stage_a_body(x, wl, wr, b, t_out, hr_out):
    xb = x[...]
    t_out[...] = jnp.concatenate(
        [jnp.dot(xb, wl[...], preferred_element_type=jnp.float32),
         jnp.ones((BN, 16), jnp.float32)], axis=1)
    hr_out[...] = jnp.dot(xb, wr[...], preferred_element_type=jnp.float32) + b[...]


def _stage_a(x, wl, wr, b):
    return pl.pallas_call(
        _stage_a_body,
        grid=(GRID,),
        in_specs=[_row_spec(DIN), _full_spec(DIN, DH), _full_spec(DIN, DH),
                  _full_spec(1, DH)],
        out_specs=[_row_spec(DH + 16), _row_spec(DH)],
        out_shape=[jax.ShapeDtypeStruct((N, DH + 16), jnp.float32),
                   jax.ShapeDtypeStruct((N, DH), jnp.float32)],
    )(x, wl, wr, b)


def _stage_b2_body(accA, accB, hr, wl, wr, b, t_out, hr_out, inv_out):
    a = accA[:, :DH] + accB[:, :DH]
    deg = accA[:, DH:DH + 1] + accB[:, DH:DH + 1]
    inv = 1.0 / jnp.maximum(deg, 1.0)
    h = jnp.maximum(a * inv + hr[...], 0.0)
    t_out[...] = jnp.dot(h, wl[...], preferred_element_type=jnp.float32)
    hr_out[...] = jnp.dot(h, wr[...], preferred_element_type=jnp.float32) + b[...]
    inv_out[...] = jnp.broadcast_to(inv, (BN, 8))


def _stage_b2(accA, accB, hr, wl, wr, b):
    return pl.pallas_call(
        _stage_b2_body,
        grid=(GRID,),
        in_specs=[_row_spec(DH + 16), _row_spec(DH + 16), _row_spec(DH),
                  _full_spec(DH, DH), _full_spec(DH, DH), _full_spec(1, DH)],
        out_specs=[_row_spec(DH), _row_spec(DH), _row_spec(8)],
        out_shape=[jax.ShapeDtypeStruct((N, DH), jnp.float32),
                   jax.ShapeDtypeStruct((N, DH), jnp.float32),
                   jax.ShapeDtypeStruct((N, 8), jnp.float32)],
    )(accA, accB, hr, wl, wr, b)


def _stage_b3_body(accA, accB, hr, inv_in, wl, wr, b, t_out, hr_out):
    a = accA[...] + accB[...]
    inv = inv_in[:, 0:1]
    h = jnp.maximum(a * inv + hr[...], 0.0)
    t_out[...] = jnp.dot(h, wl[...], preferred_element_type=jnp.float32)
    hr_out[...] = jnp.dot(h, wr[...], preferred_element_type=jnp.float32) + b[...]


def _stage_b3(accA, accB, hr, inv, wl, wr, b):
    return pl.pallas_call(
        _stage_b3_body,
        grid=(GRID,),
        in_specs=[_row_spec(DH), _row_spec(DH), _row_spec(DH), _row_spec(8),
                  _full_spec(DH, DOUT), _full_spec(DH, DOUT), _full_spec(1, DOUT)],
        out_specs=[_row_spec(DOUT), _row_spec(DOUT)],
        out_shape=[jax.ShapeDtypeStruct((N, DOUT), jnp.float32),
                   jax.ShapeDtypeStruct((N, DOUT), jnp.float32)],
    )(accA, accB, hr, inv, wl, wr, b)


def _stage_c_body(accA, accB, hr, inv_in, out):
    a = accA[...] + accB[...]
    inv = inv_in[:, 0:1]
    h = a * inv + hr[...]
    m = jnp.max(h, axis=1, keepdims=True)
    e = jnp.exp(h - m)
    lse = jnp.log(jnp.sum(e, axis=1, keepdims=True))
    out[...] = h - m - lse


def _stage_c(accA, accB, hr, inv):
    return pl.pallas_call(
        _stage_c_body,
        grid=(GRID,),
        in_specs=[_row_spec(DOUT), _row_spec(DOUT), _row_spec(DOUT), _row_spec(8)],
        out_specs=_row_spec(DOUT),
        out_shape=jax.ShapeDtypeStruct((N, DOUT), jnp.float32),
    )(accA, accB, hr, inv)


def kernel(x, edge_index, W_l1, W_r1, b1, g1, be1, W_l2, W_r2, b2, g2, be2,
           W_l3, W_r3, b3):
    s1 = g1 / jnp.sqrt(1.0 + EPS)
    s2 = g2 / jnp.sqrt(1.0 + EPS)
    wl1 = W_l1 * s1
    wr1 = W_r1 * s1
    bb1 = (b1 * s1 + be1).reshape(1, DH)
    wl2 = W_l2 * s2
    wr2 = W_r2 * s2
    bb2 = (b2 * s2 + be2).reshape(1, DH)
    bb3 = b3.reshape(1, DOUT)

    pad = E_PAD - E
    src = jnp.concatenate([edge_index[0], jnp.zeros((pad,), jnp.int32)])
    dst = jnp.concatenate([edge_index[1], jnp.full((pad,), N, jnp.int32)])
    src = src.reshape(NW, T, B)
    dst = dst.reshape(NW, T, B)

    z144 = jnp.zeros((RPT, DH + 16), jnp.float32)
    z128 = jnp.zeros((RPT, DH), jnp.float32)
    z64 = jnp.zeros((RPT, DOUT), jnp.float32)

    t1, hr1 = _stage_a(x, wl1, wr1, bb1)
    acc1 = _sc_seg_sum_144(t1, src, dst, z144)
    t2, hr2, inv = _stage_b2(acc1[0, :N], acc1[1, :N], hr1, wl2, wr2, bb2)
    acc2 = _sc_seg_sum_128(t2, src, dst, z128)
    t3, hr3 = _stage_b3(acc2[0, :N], acc2[1, :N], hr2, inv, W_l3, W_r3, bb3)
    acc3 = _sc_seg_sum_64(t3, src, dst, z64)
    return _stage_c(acc3[0, :N], acc3[1, :N], hr3, inv)


# speedup vs baseline: 3.6003x; 1.1548x over previous
import functools

import jax
import jax.numpy as jnp
from jax import lax
from jax.experimental import pallas as pl
from jax.experimental.pallas import tpu as pltpu, tpu_sc as plsc

N = 10000
E = 320000
DIN = 128
DH = 128
DOUT = 64
EPS = 1e-5

NC = 2
NS = 16
NW = NC * NS
B = 128
T = 80
E_PAD = NW * T * B
N_ROWS = 10016
RPT = N_ROWS // NS

BN = 400
GRID = N // BN


K = 2
R = T // K


def _make_sc_seg_sum(D):
    mesh = plsc.VectorSubcoreMesh(core_axis_name="c", subcore_axis_name="s")

    @functools.partial(
        pl.kernel,
        out_type=jax.ShapeDtypeStruct((NC, N_ROWS, D), jnp.float32),
        mesh=mesh,
        compiler_params=pltpu.CompilerParams(use_tc_tiling_on_sc=False),
        scratch_types=[
            pltpu.VMEM((2, K, B), jnp.int32),
            pltpu.VMEM((2, K, B), jnp.int32),
            pltpu.VMEM((K, B, D), jnp.float32),
            pltpu.VMEM_SHARED((N_ROWS, D), jnp.float32),
            pltpu.SemaphoreType.DMA((2,)),
            pltpu.SemaphoreType.DMA((2,)),
            pltpu.SemaphoreType.DMA((K,)),
            pltpu.SemaphoreType.DMA((K,)),
        ],
    )
    def seg_sum(table, src_idx, dst_idx, zeros, out,
                sib, dib, bufs, acc, isems, jsems, gsems, ssems):
        c = lax.axis_index("c")
        s = lax.axis_index("s")
        w = c * NS + s

        pltpu.async_copy(src_idx.at[w, pl.ds(0, K)], sib.at[0], isems.at[0])
        pltpu.async_copy(dst_idx.at[w, pl.ds(0, K)], dib.at[0], jsems.at[0])
        pltpu.sync_copy(zeros, acc.at[pl.ds(s * RPT, RPT)])
        plsc.subcore_barrier()

        def round_body(r, carry):
            cur = lax.rem(r, 2)
            nxt = 1 - cur

            @pl.when(r + 1 < R)
            def _prefetch():
                pltpu.async_copy(src_idx.at[w, pl.ds((r + 1) * K, K)],
                                 sib.at[nxt], isems.at[nxt])
                pltpu.async_copy(dst_idx.at[w, pl.ds((r + 1) * K, K)],
                                 dib.at[nxt], jsems.at[nxt])

            pltpu.make_async_copy(src_idx.at[w, pl.ds(0, K)], sib.at[cur],
                                  isems.at[cur]).wait()
            gd = [pltpu.async_copy(table.at[sib.at[cur, i]],
                                   bufs.at[i], gsems.at[i])
                  for i in range(K)]
            pltpu.make_async_copy(dst_idx.at[w, pl.ds(0, K)], dib.at[cur],
                                  jsems.at[cur]).wait()
            sd = []
            for i in range(K):
                gd[i].wait()
                sd.append(pltpu.async_copy(bufs.at[i],
                                           acc.at[dib.at[cur, i]],
                                           ssems.at[i], add=True))
            for i in range(K):
                sd[i].wait()
            return carry

        lax.fori_loop(0, R, round_body, 0)
        plsc.subcore_barrier()

        pltpu.sync_copy(acc.at[pl.ds(s * RPT, RPT)],
                        out.at[c, pl.ds(s * RPT, RPT)])

    return seg_sum


_sc_seg_sum_144 = _make_sc_seg_sum(DH + 16)
_sc_seg_sum_128 = _make_sc_seg_sum(DH)
_sc_seg_sum_64 = _make_sc_seg_sum(DOUT)


def _row_spec(d):
    return pl.BlockSpec((BN, d), lambda i: (i, 0))


def _full_spec(r, d):
    return pl.BlockSpec((r, d), lambda i: (0, 0))


def _stage_a_body(x, wl, wr, b, t_out, hr_out):
    xb = x[...]
    t_out[...] = jnp.concatenate(
        [jnp.dot(xb, wl[...], preferred_element_type=jnp.float32),
         jnp.ones((BN, 16), jnp.float32)], axis=1)
    hr_out[...] = jnp.dot(xb, wr[...], preferred_element_type=jnp.float32) + b[...]


def _stage_a(x, wl, wr, b):
    return pl.pallas_call(
        _stage_a_body,
        grid=(GRID,),
        in_specs=[_row_spec(DIN), _full_spec(DIN, DH), _full_spec(DIN, DH),
                  _full_spec(1, DH)],
        out_specs=[_row_spec(DH + 16), _row_spec(DH)],
        out_shape=[jax.ShapeDtypeStruct((N, DH + 16), jnp.float32),
                   jax.ShapeDtypeStruct((N, DH), jnp.float32)],
    )(x, wl, wr, b)


def _stage_b2_body(accA, accB, hr, wl, wr, b, t_out, hr_out, inv_out):
    a = accA[:, :DH] + accB[:, :DH]
    deg = accA[:, DH:DH + 1] + accB[:, DH:DH + 1]
    inv = 1.0 / jnp.maximum(deg, 1.0)
    h = jnp.maximum(a * inv + hr[...], 0.0)
    t_out[...] = jnp.dot(h, wl[...], preferred_element_type=jnp.float32)
    hr_out[...] = jnp.dot(h, wr[...], preferred_element_type=jnp.float32) + b[...]
    inv_out[...] = jnp.broadcast_to(inv, (BN, 8))


def _stage_b2(accA, accB, hr, wl, wr, b):
    return pl.pallas_call(
        _stage_b2_body,
        grid=(GRID,),
        in_specs=[_row_spec(DH + 16), _row_spec(DH + 16), _row_spec(DH),
                  _full_spec(DH, DH), _full_spec(DH, DH), _full_spec(1, DH)],
        out_specs=[_row_spec(DH), _row_spec(DH), _row_spec(8)],
        out_shape=[jax.ShapeDtypeStruct((N, DH), jnp.float32),
                   jax.ShapeDtypeStruct((N, DH), jnp.float32),
                   jax.ShapeDtypeStruct((N, 8), jnp.float32)],
    )(accA, accB, hr, wl, wr, b)


def _stage_b3_body(accA, accB, hr, inv_in, wl, wr, b, t_out, hr_out):
    a = accA[...] + accB[...]
    inv = inv_in[:, 0:1]
    h = jnp.maximum(a * inv + hr[...], 0.0)
    t_out[...] = jnp.dot(h, wl[...], preferred_element_type=jnp.float32)
    hr_out[...] = jnp.dot(h, wr[...], preferred_element_type=jnp.float32) + b[...]


def _stage_b3(accA, accB, hr, inv, wl, wr, b):
    return pl.pallas_call(
        _stage_b3_body,
        grid=(GRID,),
        in_specs=[_row_spec(DH), _row_spec(DH), _row_spec(DH), _row_spec(8),
                  _full_spec(DH, DOUT), _full_spec(DH, DOUT), _full_spec(1, DOUT)],
        out_specs=[_row_spec(DOUT), _row_spec(DOUT)],
        out_shape=[jax.ShapeDtypeStruct((N, DOUT), jnp.float32),
                   jax.ShapeDtypeStruct((N, DOUT), jnp.float32)],
    )(accA, accB, hr, inv, wl, wr, b)


def _stage_c_body(accA, accB, hr, inv_in, out):
    a = accA[...] + accB[...]
    inv = inv_in[:, 0:1]
    h = a * inv + hr[...]
    m = jnp.max(h, axis=1, keepdims=True)
    e = jnp.exp(h - m)
    lse = jnp.log(jnp.sum(e, axis=1, keepdims=True))
    out[...] = h - m - lse


def _stage_c(accA, accB, hr, inv):
    return pl.pallas_call(
        _stage_c_body,
        grid=(GRID,),
        in_specs=[_row_spec(DOUT), _row_spec(DOUT), _row_spec(DOUT), _row_spec(8)],
        out_specs=_row_spec(DOUT),
        out_shape=jax.ShapeDtypeStruct((N, DOUT), jnp.float32),
    )(accA, accB, hr, inv)


def kernel(x, edge_index, W_l1, W_r1, b1, g1, be1, W_l2, W_r2, b2, g2, be2,
           W_l3, W_r3, b3):
    s1 = g1 / jnp.sqrt(1.0 + EPS)
    s2 = g2 / jnp.sqrt(1.0 + EPS)
    wl1 = W_l1 * s1
    wr1 = W_r1 * s1
    bb1 = (b1 * s1 + be1).reshape(1, DH)
    wl2 = W_l2 * s2
    wr2 = W_r2 * s2
    bb2 = (b2 * s2 + be2).reshape(1, DH)
    bb3 = b3.reshape(1, DOUT)

    pad = E_PAD - E
    src = jnp.concatenate([edge_index[0], jnp.zeros((pad,), jnp.int32)])
    dst = jnp.concatenate([edge_index[1], jnp.full((pad,), N, jnp.int32)])
    src = src.reshape(NW, T, B)
    dst = dst.reshape(NW, T, B)

    z144 = jnp.zeros((RPT, DH + 16), jnp.float32)
    z128 = jnp.zeros((RPT, DH), jnp.float32)
    z64 = jnp.zeros((RPT, DOUT), jnp.float32)

    t1, hr1 = _stage_a(x, wl1, wr1, bb1)
    acc1 = _sc_seg_sum_144(t1, src, dst, z144)
    t2, hr2, inv = _stage_b2(acc1[0, :N], acc1[1, :N], hr1, wl2, wr2, bb2)
    acc2 = _sc_seg_sum_128(t2, src, dst, z128)
    t3, hr3 = _stage_b3(acc2[0, :N], acc2[1, :N], hr2, inv, W_l3, W_r3, bb3)
    acc3 = _sc_seg_sum_64(t3, src, dst, z64)
    return _stage_c(acc3[0, :N], acc3[1, :N], hr3, inv)


# speedup vs baseline: 8.8809x; 2.4667x over previous
import functools

import jax
import jax.numpy as jnp
from jax import lax
from jax.experimental import pallas as pl
from jax.experimental.pallas import tpu as pltpu, tpu_sc as plsc

N = 10000
E = 320000
DIN = 128
DH = 128
DOUT = 64
EPS = 1e-5

NC = 2
NS = 16
NW = NC * NS
B = 125
T = 80
N_ROWS = 10016
RPT = N_ROWS // NS

BN = 400
GRID = N // BN


K = 2
R = T // K


def _make_sc_seg_sum(D):
    mesh = plsc.VectorSubcoreMesh(core_axis_name="c", subcore_axis_name="s")

    @functools.partial(
        pl.kernel,
        out_type=jax.ShapeDtypeStruct((NC, N_ROWS, D), jnp.float32),
        mesh=mesh,
        compiler_params=pltpu.CompilerParams(use_tc_tiling_on_sc=False),
        scratch_types=[
            pltpu.VMEM((2, K, B), jnp.int32),
            pltpu.VMEM((2, K, B), jnp.int32),
            pltpu.VMEM((K, B, D), jnp.float32),
            pltpu.VMEM_SHARED((N_ROWS, D), jnp.float32),
            pltpu.SemaphoreType.DMA((2,)),
            pltpu.SemaphoreType.DMA((2,)),
            pltpu.SemaphoreType.DMA((K,)),
            pltpu.SemaphoreType.DMA((K,)),
        ],
    )
    def seg_sum(table, src_idx, dst_idx, zeros, out,
                sib, dib, bufs, acc, isems, jsems, gsems, ssems):
        c = lax.axis_index("c")
        s = lax.axis_index("s")
        w = c * NS + s

        pltpu.async_copy(src_idx.at[w, 0], sib.at[0], isems.at[0])
        pltpu.async_copy(dst_idx.at[w, 0], dib.at[0], jsems.at[0])
        pltpu.sync_copy(zeros, acc.at[pl.ds(s * RPT, RPT)])
        plsc.subcore_barrier()

        def round_body(r, carry):
            cur = lax.rem(r, 2)
            nxt = 1 - cur

            @pl.when(r + 1 < R)
            def _prefetch():
                pltpu.async_copy(src_idx.at[w, r + 1], sib.at[nxt],
                                 isems.at[nxt])
                pltpu.async_copy(dst_idx.at[w, r + 1], dib.at[nxt],
                                 jsems.at[nxt])

            pltpu.make_async_copy(src_idx.at[w, 0], sib.at[cur],
                                  isems.at[cur]).wait()
            gd = [pltpu.async_copy(table.at[sib.at[cur, i]],
                                   bufs.at[i], gsems.at[i])
                  for i in range(K)]
            pltpu.make_async_copy(dst_idx.at[w, 0], dib.at[cur],
                                  jsems.at[cur]).wait()
            sd = []
            for i in range(K):
                gd[i].wait()
                sd.append(pltpu.async_copy(bufs.at[i],
                                           acc.at[dib.at[cur, i]],
                                           ssems.at[i], add=True))
            for i in range(K):
                sd[i].wait()
            return carry

        lax.fori_loop(0, R, round_body, 0)
        plsc.subcore_barrier()

        pltpu.sync_copy(acc.at[pl.ds(s * RPT, RPT)],
                        out.at[c, pl.ds(s * RPT, RPT)])

    return seg_sum


_sc_seg_sum_144 = _make_sc_seg_sum(DH + 16)
_sc_seg_sum_128 = _make_sc_seg_sum(DH)
_sc_seg_sum_64 = _make_sc_seg_sum(DOUT)


def _row_spec(d):
    return pl.BlockSpec((BN, d), lambda i: (i, 0))


def _full_spec(r, d):
    return pl.BlockSpec((r, d), lambda i: (0, 0))


def _stage_a_body(x, wl, wr, b, t_out, hr_out):
    xb = x[...]
    t_out[...] = jnp.concatenate(
        [jnp.dot(xb, wl[...], preferred_element_type=jnp.float32),
         jnp.ones((BN, 16), jnp.float32)], axis=1)
    hr_out[...] = jnp.dot(xb, wr[...], preferred_element_type=jnp.float32) + b[...]


def _stage_a(x, wl, wr, b):
    return pl.pallas_call(
        _stage_a_body,
        grid=(GRID,),
        in_specs=[_row_spec(DIN), _full_spec(DIN, DH), _full_spec(DIN, DH),
                  _full_spec(1, DH)],
        out_specs=[_row_spec(DH + 16), _row_spec(DH)],
        out_shape=[jax.ShapeDtypeStruct((N, DH + 16), jnp.float32),
                   jax.ShapeDtypeStruct((N, DH), jnp.float32)],
    )(x, wl, wr, b)


def _stage_b2_body(accA, accB, hr, wl, wr, b, t_out, hr_out, inv_out):
    a = accA[:, :DH] + accB[:, :DH]
    deg = accA[:, DH:DH + 1] + accB[:, DH:DH + 1]
    inv = 1.0 / jnp.maximum(deg, 1.0)
    h = jnp.maximum(a * inv + hr[...], 0.0)
    t_out[...] = jnp.dot(h, wl[...], preferred_element_type=jnp.float32)
    hr_out[...] = jnp.dot(h, wr[...], preferred_element_type=jnp.float32) + b[...]
    inv_out[...] = jnp.broadcast_to(inv, (BN, 8))


def _stage_b2(accA, accB, hr, wl, wr, b):
    return pl.pallas_call(
        _stage_b2_body,
        grid=(GRID,),
        in_specs=[_row_spec(DH + 16), _row_spec(DH + 16), _row_spec(DH),
                  _full_spec(DH, DH), _full_spec(DH, DH), _full_spec(1, DH)],
        out_specs=[_row_spec(DH), _row_spec(DH), _row_spec(8)],
        out_shape=[jax.ShapeDtypeStruct((N, DH), jnp.float32),
                   jax.ShapeDtypeStruct((N, DH), jnp.float32),
                   jax.ShapeDtypeStruct((N, 8), jnp.float32)],
    )(accA, accB, hr, wl, wr, b)


def _stage_b3_body(accA, accB, hr, inv_in, wl, wr, b, t_out, hr_out):
    a = accA[...] + accB[...]
    inv = inv_in[:, 0:1]
    h = jnp.maximum(a * inv + hr[...], 0.0)
    t_out[...] = jnp.dot(h, wl[...], preferred_element_type=jnp.float32)
    hr_out[...] = jnp.dot(h, wr[...], preferred_element_type=jnp.float32) + b[...]


def _stage_b3(accA, accB, hr, inv, wl, wr, b):
    return pl.pallas_call(
        _stage_b3_body,
        grid=(GRID,),
        in_specs=[_row_spec(DH), _row_spec(DH), _row_spec(DH), _row_spec(8),
                  _full_spec(DH, DOUT), _full_spec(DH, DOUT), _full_spec(1, DOUT)],
        out_specs=[_row_spec(DOUT), _row_spec(DOUT)],
        out_shape=[jax.ShapeDtypeStruct((N, DOUT), jnp.float32),
                   jax.ShapeDtypeStruct((N, DOUT), jnp.float32)],
    )(accA, accB, hr, inv, wl, wr, b)


def _stage_c_body(accA, accB, hr, inv_in, out):
    a = accA[...] + accB[...]
    inv = inv_in[:, 0:1]
    h = a * inv + hr[...]
    m = jnp.max(h, axis=1, keepdims=True)
    e = jnp.exp(h - m)
    lse = jnp.log(jnp.sum(e, axis=1, keepdims=True))
    out[...] = h - m - lse


def _stage_c(accA, accB, hr, inv):
    return pl.pallas_call(
        _stage_c_body,
        grid=(GRID,),
        in_specs=[_row_spec(DOUT), _row_spec(DOUT), _row_spec(DOUT), _row_spec(8)],
        out_specs=_row_spec(DOUT),
        out_shape=jax.ShapeDtypeStruct((N, DOUT), jnp.float32),
    )(accA, accB, hr, inv)


def kernel(x, edge_index, W_l1, W_r1, b1, g1, be1, W_l2, W_r2, b2, g2, be2,
           W_l3, W_r3, b3):
    s1 = g1 / jnp.sqrt(1.0 + EPS)
    s2 = g2 / jnp.sqrt(1.0 + EPS)
    wl1 = W_l1 * s1
    wr1 = W_r1 * s1
    bb1 = (b1 * s1 + be1).reshape(1, DH)
    wl2 = W_l2 * s2
    wr2 = W_r2 * s2
    bb2 = (b2 * s2 + be2).reshape(1, DH)
    bb3 = b3.reshape(1, DOUT)

    src = edge_index[0].reshape(NW, R, K, B)
    dst = edge_index[1].reshape(NW, R, K, B)

    z144 = jnp.zeros((RPT, DH + 16), jnp.float32)
    z128 = jnp.zeros((RPT, DH), jnp.float32)
    z64 = jnp.zeros((RPT, DOUT), jnp.float32)

    t1, hr1 = _stage_a(x, wl1, wr1, bb1)
    acc1 = _sc_seg_sum_144(t1, src, dst, z144)
    t2, hr2, inv = _stage_b2(acc1[0, :N], acc1[1, :N], hr1, wl2, wr2, bb2)
    acc2 = _sc_seg_sum_128(t2, src, dst, z128)
    t3, hr3 = _stage_b3(acc2[0, :N], acc2[1, :N], hr2, inv, W_l3, W_r3, bb3)
    acc3 = _sc_seg_sum_64(t3, src, dst, z64)
    return _stage_c(acc3[0, :N], acc3[1, :N], hr3, inv)


# speedup vs baseline: 9.8561x; 1.1098x over previous
import functools

import jax
import jax.numpy as jnp
from jax import lax
from jax.experimental import pallas as pl
from jax.experimental.pallas import tpu as pltpu, tpu_sc as plsc

N = 10000
E = 320000
DIN = 128
DH = 128
DOUT = 64
EPS = 1e-5

NC = 2
NS = 16
NW = NC * NS
B = 125
T = 80
N_ROWS = 10016
RPT = N_ROWS // NS

BN = 400
GRID = N // BN


K = 2
R = T // K


DDEG = 16


def _make_sc_seg_sum(D, with_deg):
    mesh = plsc.VectorSubcoreMesh(core_axis_name="c", subcore_axis_name="s")

    out_type = [jax.ShapeDtypeStruct((NC, N_ROWS, D), jnp.float32)]
    scratch = [
        pltpu.VMEM((2, K, B), jnp.int32),
        pltpu.VMEM((2, K, B), jnp.int32),
        pltpu.VMEM((K, B, D), jnp.float32),
        pltpu.VMEM_SHARED((N_ROWS, D), jnp.float32),
        pltpu.SemaphoreType.DMA((2,)),
        pltpu.SemaphoreType.DMA((2,)),
        pltpu.SemaphoreType.DMA((K,)),
        pltpu.SemaphoreType.DMA((K,)),
    ]
    if with_deg:
        out_type.append(jax.ShapeDtypeStruct((NC, N_ROWS, DDEG), jnp.float32))
        scratch += [
            pltpu.VMEM((B, DDEG), jnp.float32),
            pltpu.VMEM_SHARED((N_ROWS, DDEG), jnp.float32),
            pltpu.SemaphoreType.DMA((K,)),
        ]

    def seg_sum(table, src_idx, dst_idx, zeros, *rest):
        if with_deg:
            (ones_hbm, z16, out, deg_out, sib, dib, bufs, acc,
             isems, jsems, gsems, ssems, ones_v, dacc, dsems) = rest
        else:
            (out, sib, dib, bufs, acc,
             isems, jsems, gsems, ssems) = rest
        c = lax.axis_index("c")
        s = lax.axis_index("s")
        w = c * NS + s

        pltpu.async_copy(src_idx.at[w, 0], sib.at[0], isems.at[0])
        pltpu.async_copy(dst_idx.at[w, 0], dib.at[0], jsems.at[0])
        pltpu.sync_copy(zeros, acc.at[pl.ds(s * RPT, RPT)])
        if with_deg:
            pltpu.sync_copy(ones_hbm, ones_v)
            pltpu.sync_copy(z16, dacc.at[pl.ds(s * RPT, RPT)])
        plsc.subcore_barrier()

        def round_body(r, carry):
            cur = lax.rem(r, 2)
            nxt = 1 - cur

            @pl.when(r + 1 < R)
            def _prefetch():
                pltpu.async_copy(src_idx.at[w, r + 1], sib.at[nxt],
                                 isems.at[nxt])
                pltpu.async_copy(dst_idx.at[w, r + 1], dib.at[nxt],
                                 jsems.at[nxt])

            pltpu.make_async_copy(src_idx.at[w, 0], sib.at[cur],
                                  isems.at[cur]).wait()
            gd = [pltpu.async_copy(table.at[sib.at[cur, i]],
                                   bufs.at[i], gsems.at[i])
                  for i in range(K)]
            pltpu.make_async_copy(dst_idx.at[w, 0], dib.at[cur],
                                  jsems.at[cur]).wait()
            sd = []
            for i in range(K):
                gd[i].wait()
                sd.append(pltpu.async_copy(bufs.at[i],
                                           acc.at[dib.at[cur, i]],
                                           ssems.at[i], add=True))
                if with_deg:
                    sd.append(pltpu.async_copy(ones_v,
                                               dacc.at[dib.at[cur, i]],
                                               dsems.at[i], add=True))
            for d in sd:
                d.wait()
            return carry

        lax.fori_loop(0, R, round_body, 0)
        plsc.subcore_barrier()

        pltpu.sync_copy(acc.at[pl.ds(s * RPT, RPT)],
                        out.at[c, pl.ds(s * RPT, RPT)])
        if with_deg:
            pltpu.sync_copy(dacc.at[pl.ds(s * RPT, RPT)],
                            deg_out.at[c, pl.ds(s * RPT, RPT)])

    return pl.kernel(
        seg_sum,
        out_type=out_type if len(out_type) > 1 else out_type[0],
        mesh=mesh,
        compiler_params=pltpu.CompilerParams(use_tc_tiling_on_sc=False),
        scratch_types=scratch,
    )


_sc_seg_sum_deg = _make_sc_seg_sum(DH, with_deg=True)
_sc_seg_sum_128 = _make_sc_seg_sum(DH, with_deg=False)
_sc_seg_sum_64 = _make_sc_seg_sum(DOUT, with_deg=False)


def _row_spec(d):
    return pl.BlockSpec((BN, d), lambda i: (i, 0))


def _acc_spec(part, d):
    return pl.BlockSpec((1, BN, d), lambda i, _p=part: (_p, i, 0))


def _full_spec(r, d):
    return pl.BlockSpec((r, d), lambda i: (0, 0))


def _stage_a_body(x, wl, wr, b, t_out, hr_out):
    xb = x[...]
    t_out[...] = jnp.dot(xb, wl[...], preferred_element_type=jnp.float32)
    hr_out[...] = jnp.dot(xb, wr[...], preferred_element_type=jnp.float32) + b[...]


def _stage_a(x, wl, wr, b):
    return pl.pallas_call(
        _stage_a_body,
        grid=(GRID,),
        in_specs=[_row_spec(DIN), _full_spec(DIN, DH), _full_spec(DIN, DH),
                  _full_spec(1, DH)],
        out_specs=[_row_spec(DH), _row_spec(DH)],
        out_shape=[jax.ShapeDtypeStruct((N, DH), jnp.float32),
                   jax.ShapeDtypeStruct((N, DH), jnp.float32)],
    )(x, wl, wr, b)


def _stage_b2_body(accA, accB, degA, degB, hr, wl, wr, b,
                   t_out, hr_out, inv_out):
    a = accA[0] + accB[0]
    deg = degA[0, :, 0:1] + degB[0, :, 0:1]
    inv = 1.0 / jnp.maximum(deg, 1.0)
    h = jnp.maximum(a * inv + hr[...], 0.0)
    t_out[...] = jnp.dot(h, wl[...], preferred_element_type=jnp.float32)
    hr_out[...] = jnp.dot(h, wr[...], preferred_element_type=jnp.float32) + b[...]
    inv_out[...] = jnp.broadcast_to(inv, (BN, 8))


def _stage_b2(acc, deg, hr, wl, wr, b):
    return pl.pallas_call(
        _stage_b2_body,
        grid=(GRID,),
        in_specs=[_acc_spec(0, DH), _acc_spec(1, DH),
                  _acc_spec(0, DDEG), _acc_spec(1, DDEG), _row_spec(DH),
                  _full_spec(DH, DH), _full_spec(DH, DH), _full_spec(1, DH)],
        out_specs=[_row_spec(DH), _row_spec(DH), _row_spec(8)],
        out_shape=[jax.ShapeDtypeStruct((N, DH), jnp.float32),
                   jax.ShapeDtypeStruct((N, DH), jnp.float32),
                   jax.ShapeDtypeStruct((N, 8), jnp.float32)],
    )(acc, acc, deg, deg, hr, wl, wr, b)


def _stage_b3_body(accA, accB, hr, inv_in, wl, wr, b, t_out, hr_out):
    a = accA[0] + accB[0]
    inv = inv_in[:, 0:1]
    h = jnp.maximum(a * inv + hr[...], 0.0)
    t_out[...] = jnp.dot(h, wl[...], preferred_element_type=jnp.float32)
    hr_out[...] = jnp.dot(h, wr[...], preferred_element_type=jnp.float32) + b[...]


def _stage_b3(acc, hr, inv, wl, wr, b):
    return pl.pallas_call(
        _stage_b3_body,
        grid=(GRID,),
        in_specs=[_acc_spec(0, DH), _acc_spec(1, DH), _row_spec(DH),
                  _row_spec(8),
                  _full_spec(DH, DOUT), _full_spec(DH, DOUT), _full_spec(1, DOUT)],
        out_specs=[_row_spec(DOUT), _row_spec(DOUT)],
        out_shape=[jax.ShapeDtypeStruct((N, DOUT), jnp.float32),
                   jax.ShapeDtypeStruct((N, DOUT), jnp.float32)],
    )(acc, acc, hr, inv, wl, wr, b)


def _stage_c_body(accA, accB, hr, inv_in, out):
    a = accA[0] + accB[0]
    inv = inv_in[:, 0:1]
    h = a * inv + hr[...]
    m = jnp.max(h, axis=1, keepdims=True)
    e = jnp.exp(h - m)
    lse = jnp.log(jnp.sum(e, axis=1, keepdims=True))
    out[...] = h - m - lse


def _stage_c(acc, hr, inv):
    return pl.pallas_call(
        _stage_c_body,
        grid=(GRID,),
        in_specs=[_acc_spec(0, DOUT), _acc_spec(1, DOUT), _row_spec(DOUT),
                  _row_spec(8)],
        out_specs=_row_spec(DOUT),
        out_shape=jax.ShapeDtypeStruct((N, DOUT), jnp.float32),
    )(acc, acc, hr, inv)


def kernel(x, edge_index, W_l1, W_r1, b1, g1, be1, W_l2, W_r2, b2, g2, be2,
           W_l3, W_r3, b3):
    s1 = g1 / jnp.sqrt(1.0 + EPS)
    s2 = g2 / jnp.sqrt(1.0 + EPS)
    wl1 = W_l1 * s1
    wr1 = W_r1 * s1
    bb1 = (b1 * s1 + be1).reshape(1, DH)
    wl2 = W_l2 * s2
    wr2 = W_r2 * s2
    bb2 = (b2 * s2 + be2).reshape(1, DH)
    bb3 = b3.reshape(1, DOUT)

    src = edge_index[0].reshape(NW, R, K, B)
    dst = edge_index[1].reshape(NW, R, K, B)

    z128 = jnp.zeros((RPT, DH), jnp.float32)
    z64 = jnp.zeros((RPT, DOUT), jnp.float32)
    z16 = jnp.zeros((RPT, DDEG), jnp.float32)
    ones16 = jnp.ones((B, DDEG), jnp.float32)

    t1, hr1 = _stage_a(x, wl1, wr1, bb1)
    acc1, deg = _sc_seg_sum_deg(t1, src, dst, z128, ones16, z16)
    t2, hr2, inv = _stage_b2(acc1, deg, hr1, wl2, wr2, bb2)
    acc2 = _sc_seg_sum_128(t2, src, dst, z128)
    t3, hr3 = _stage_b3(acc2, hr2, inv, W_l3, W_r3, bb3)
    acc3 = _sc_seg_sum_64(t3, src, dst, z64)
    return _stage_c(acc3, hr3, inv)


# speedup vs baseline: 13.0775x; 1.3268x over previous
import functools

import jax
import jax.numpy as jnp
from jax import lax
from jax.experimental import pallas as pl
from jax.experimental.pallas import tpu as pltpu, tpu_sc as plsc

N = 10000
E = 320000
DIN = 128
DH = 128
DOUT = 64
EPS = 1e-5

NC = 2
NS = 16
NW = NC * NS
B = 100
T = 100
N_ROWS = 10016
RPT = N_ROWS // NS

BN = 2000
GRID = N // BN


R = T


DDEG = 16


def _make_sc_seg_sum(D, with_deg):
    mesh = plsc.VectorSubcoreMesh(core_axis_name="c", subcore_axis_name="s")

    out_type = [jax.ShapeDtypeStruct((NC, N_ROWS, D), jnp.float32)]
    scratch = [
        pltpu.VMEM((4, B), jnp.int32),
        pltpu.VMEM((4, B), jnp.int32),
        pltpu.VMEM((2, B, D), jnp.float32),
        pltpu.VMEM_SHARED((N_ROWS, D), jnp.float32),
        pltpu.SemaphoreType.DMA((4,)),
        pltpu.SemaphoreType.DMA((4,)),
        pltpu.SemaphoreType.DMA((2,)),
        pltpu.SemaphoreType.DMA((2,)),
    ]
    if with_deg:
        out_type.append(jax.ShapeDtypeStruct((NC, N_ROWS, DDEG), jnp.float32))
        scratch += [
            pltpu.VMEM((B, DDEG), jnp.float32),
            pltpu.VMEM_SHARED((N_ROWS, DDEG), jnp.float32),
            pltpu.SemaphoreType.DMA((2,)),
        ]

    def seg_sum(table, src_idx, dst_idx, zeros, *rest):
        if with_deg:
            (ones_hbm, z16, out, deg_out, sib, dib, bufs, acc,
             isems, jsems, gsems, ssems, ones_v, dacc, dsems) = rest
        else:
            (out, sib, dib, bufs, acc,
             isems, jsems, gsems, ssems) = rest
        c = lax.axis_index("c")
        s = lax.axis_index("s")
        w = c * NS + s

        pltpu.async_copy(src_idx.at[w, 0], sib.at[0], isems.at[0])
        pltpu.async_copy(dst_idx.at[w, 0], dib.at[0], jsems.at[0])
        pltpu.sync_copy(zeros, acc.at[pl.ds(s * RPT, RPT)])
        if with_deg:
            pltpu.sync_copy(ones_hbm, ones_v)
            pltpu.sync_copy(z16, dacc.at[pl.ds(s * RPT, RPT)])
        plsc.subcore_barrier()

        def round_body(r, carry):
            slot = lax.rem(r, 4)
            st = lax.rem(r, 2)
            oslot = lax.rem(r + 3, 4)
            ost = 1 - st

            @pl.when(r >= 2)
            def _drain():
                dslot = lax.rem(r + 2, 4)
                pltpu.make_async_copy(bufs.at[st], acc.at[dib.at[dslot]],
                                      ssems.at[st]).wait()
                if with_deg:
                    pltpu.make_async_copy(ones_v, dacc.at[dib.at[dslot]],
                                          dsems.at[st]).wait()

            @pl.when(r < R)
            def _gather():
                @pl.when(r + 1 < R)
                def _prefetch():
                    nslot = lax.rem(r + 1, 4)
                    pltpu.async_copy(src_idx.at[w, r + 1], sib.at[nslot],
                                     isems.at[nslot])
                    pltpu.async_copy(dst_idx.at[w, r + 1], dib.at[nslot],
                                     jsems.at[nslot])
                pltpu.make_async_copy(src_idx.at[w, 0], sib.at[slot],
                                      isems.at[slot]).wait()
                pltpu.async_copy(table.at[sib.at[slot]], bufs.at[st],
                                 gsems.at[st])

            @pl.when((r >= 1) & (r <= R))
            def _scatter():
                pltpu.make_async_copy(table.at[sib.at[oslot]], bufs.at[ost],
                                      gsems.at[ost]).wait()
                pltpu.make_async_copy(dst_idx.at[w, 0], dib.at[oslot],
                                      jsems.at[oslot]).wait()
                pltpu.async_copy(bufs.at[ost], acc.at[dib.at[oslot]],
                                 ssems.at[ost], add=True)
                if with_deg:
                    pltpu.async_copy(ones_v, dacc.at[dib.at[oslot]],
                                     dsems.at[ost], add=True)

            return carry

        lax.fori_loop(0, R + 2, round_body, 0)
        plsc.subcore_barrier()

        pltpu.sync_copy(acc.at[pl.ds(s * RPT, RPT)],
                        out.at[c, pl.ds(s * RPT, RPT)])
        if with_deg:
            pltpu.sync_copy(dacc.at[pl.ds(s * RPT, RPT)],
                            deg_out.at[c, pl.ds(s * RPT, RPT)])

    return pl.kernel(
        seg_sum,
        out_type=out_type if len(out_type) > 1 else out_type[0],
        mesh=mesh,
        compiler_params=pltpu.CompilerParams(use_tc_tiling_on_sc=False),
        scratch_types=scratch,
    )


_sc_seg_sum_deg = _make_sc_seg_sum(DH, with_deg=True)
_sc_seg_sum_128 = _make_sc_seg_sum(DH, with_deg=False)
_sc_seg_sum_64 = _make_sc_seg_sum(DOUT, with_deg=False)


def _row_spec(d):
    return pl.BlockSpec((BN, d), lambda i: (i, 0))


def _acc_spec(part, d):
    return pl.BlockSpec((1, BN, d), lambda i, _p=part: (_p, i, 0))


def _full_spec(r, d):
    return pl.BlockSpec((r, d), lambda i: (0, 0))


def _stage_a_body(x, wl, wr, b, t_out, hr_out):
    xb = x[...]
    t_out[...] = jnp.dot(xb, wl[...], preferred_element_type=jnp.float32)
    hr_out[...] = jnp.dot(xb, wr[...], preferred_element_type=jnp.float32) + b[...]


def _stage_a(x, wl, wr, b):
    return pl.pallas_call(
        _stage_a_body,
        grid=(GRID,),
        in_specs=[_row_spec(DIN), _full_spec(DIN, DH), _full_spec(DIN, DH),
                  _full_spec(1, DH)],
        out_specs=[_row_spec(DH), _row_spec(DH)],
        out_shape=[jax.ShapeDtypeStruct((N, DH), jnp.float32),
                   jax.ShapeDtypeStruct((N, DH), jnp.float32)],
    )(x, wl, wr, b)


def _stage_b2_body(accA, accB, degA, degB, hr, wl, wr, b,
                   t_out, hr_out, inv_out):
    a = accA[0] + accB[0]
    deg = degA[0, :, 0:1] + degB[0, :, 0:1]
    inv = 1.0 / jnp.maximum(deg, 1.0)
    h = jnp.maximum(a * inv + hr[...], 0.0)
    t_out[...] = jnp.dot(h, wl[...], preferred_element_type=jnp.float32)
    hr_out[...] = jnp.dot(h, wr[...], preferred_element_type=jnp.float32) + b[...]
    inv_out[...] = jnp.broadcast_to(inv, (BN, 8))


def _stage_b2(acc, deg, hr, wl, wr, b):
    return pl.pallas_call(
        _stage_b2_body,
        grid=(GRID,),
        in_specs=[_acc_spec(0, DH), _acc_spec(1, DH),
                  _acc_spec(0, DDEG), _acc_spec(1, DDEG), _row_spec(DH),
                  _full_spec(DH, DH), _full_spec(DH, DH), _full_spec(1, DH)],
        out_specs=[_row_spec(DH), _row_spec(DH), _row_spec(8)],
        out_shape=[jax.ShapeDtypeStruct((N, DH), jnp.float32),
                   jax.ShapeDtypeStruct((N, DH), jnp.float32),
                   jax.ShapeDtypeStruct((N, 8), jnp.float32)],
    )(acc, acc, deg, deg, hr, wl, wr, b)


def _stage_b3_body(accA, accB, hr, inv_in, wl, wr, b, t_out, hr_out):
    a = accA[0] + accB[0]
    inv = inv_in[:, 0:1]
    h = jnp.maximum(a * inv + hr[...], 0.0)
    t_out[...] = jnp.dot(h, wl[...], preferred_element_type=jnp.float32)
    hr_out[...] = jnp.dot(h, wr[...], preferred_element_type=jnp.float32) + b[...]


def _stage_b3(acc, hr, inv, wl, wr, b):
    return pl.pallas_call(
        _stage_b3_body,
        grid=(GRID,),
        in_specs=[_acc_spec(0, DH), _acc_spec(1, DH), _row_spec(DH),
                  _row_spec(8),
                  _full_spec(DH, DOUT), _full_spec(DH, DOUT), _full_spec(1, DOUT)],
        out_specs=[_row_spec(DOUT), _row_spec(DOUT)],
        out_shape=[jax.ShapeDtypeStruct((N, DOUT), jnp.float32),
                   jax.ShapeDtypeStruct((N, DOUT), jnp.float32)],
    )(acc, acc, hr, inv, wl, wr, b)


def _stage_c_body(accA, accB, hr, inv_in, out):
    a = accA[0] + accB[0]
    inv = inv_in[:, 0:1]
    h = a * inv + hr[...]
    m = jnp.max(h, axis=1, keepdims=True)
    e = jnp.exp(h - m)
    lse = jnp.log(jnp.sum(e, axis=1, keepdims=True))
    out[...] = h - m - lse


def _stage_c(acc, hr, inv):
    return pl.pallas_call(
        _stage_c_body,
        grid=(GRID,),
        in_specs=[_acc_spec(0, DOUT), _acc_spec(1, DOUT), _row_spec(DOUT),
                  _row_spec(8)],
        out_specs=_row_spec(DOUT),
        out_shape=jax.ShapeDtypeStruct((N, DOUT), jnp.float32),
    )(acc, acc, hr, inv)


def kernel(x, edge_index, W_l1, W_r1, b1, g1, be1, W_l2, W_r2, b2, g2, be2,
           W_l3, W_r3, b3):
    s1 = g1 / jnp.sqrt(1.0 + EPS)
    s2 = g2 / jnp.sqrt(1.0 + EPS)
    wl1 = W_l1 * s1
    wr1 = W_r1 * s1
    bb1 = (b1 * s1 + be1).reshape(1, DH)
    wl2 = W_l2 * s2
    wr2 = W_r2 * s2
    bb2 = (b2 * s2 + be2).reshape(1, DH)
    bb3 = b3.reshape(1, DOUT)

    src = edge_index[0].reshape(NW, T, B)
    dst = edge_index[1].reshape(NW, T, B)

    z128 = jnp.zeros((RPT, DH), jnp.float32)
    z64 = jnp.zeros((RPT, DOUT), jnp.float32)
    z16 = jnp.zeros((RPT, DDEG), jnp.float32)
    ones16 = jnp.ones((B, DDEG), jnp.float32)

    t1, hr1 = _stage_a(x, wl1, wr1, bb1)
    acc1, deg = _sc_seg_sum_deg(t1, src, dst, z128, ones16, z16)
    t2, hr2, inv = _stage_b2(acc1, deg, hr1, wl2, wr2, bb2)
    acc2 = _sc_seg_sum_128(t2, src, dst, z128)
    t3, hr3 = _stage_b3(acc2, hr2, inv, W_l3, W_r3, bb3)
    acc3 = _sc_seg_sum_64(t3, src, dst, z64)
    return _stage_c(acc3, hr3, inv)


# speedup vs baseline: 13.5722x; 1.0378x over previous
import functools

import jax
import jax.numpy as jnp
from jax import lax
from jax.experimental import pallas as pl
from jax.experimental.pallas import tpu as pltpu, tpu_sc as plsc

N = 10000
E = 320000
DIN = 128
DH = 128
DOUT = 64
EPS = 1e-5

NC = 2
NS = 16
NW = NC * NS
B = 125
T = 80
N_ROWS = 10016
RPT = N_ROWS // NS

BN = 2000
GRID = N // BN


R = T


DDEG = 16


def _make_sc_seg_sum(D, with_deg):
    mesh = plsc.VectorSubcoreMesh(core_axis_name="c", subcore_axis_name="s")

    out_type = [jax.ShapeDtypeStruct((NC, N_ROWS, D), jnp.float32)]
    scratch = [
        pltpu.VMEM((4, B), jnp.int32),
        pltpu.VMEM((4, B), jnp.int32),
        pltpu.VMEM((2, B, D), jnp.float32),
        pltpu.VMEM_SHARED((N_ROWS, D), jnp.float32),
        pltpu.SemaphoreType.DMA((4,)),
        pltpu.SemaphoreType.DMA((4,)),
        pltpu.SemaphoreType.DMA((2,)),
        pltpu.SemaphoreType.DMA((2,)),
    ]
    if with_deg:
        out_type.append(jax.ShapeDtypeStruct((NC, N_ROWS, DDEG), jnp.float32))
        scratch += [
            pltpu.VMEM((B, DDEG), jnp.float32),
            pltpu.VMEM_SHARED((N_ROWS, DDEG), jnp.float32),
            pltpu.SemaphoreType.DMA((2,)),
        ]

    def seg_sum(table, src_idx, dst_idx, zeros, *rest):
        if with_deg:
            (ones_hbm, z16, out, deg_out, sib, dib, bufs, acc,
             isems, jsems, gsems, ssems, ones_v, dacc, dsems) = rest
        else:
            (out, sib, dib, bufs, acc,
             isems, jsems, gsems, ssems) = rest
        c = lax.axis_index("c")
        s = lax.axis_index("s")
        w = c * NS + s

        pltpu.async_copy(src_idx.at[w, 0], sib.at[0], isems.at[0])
        pltpu.async_copy(dst_idx.at[w, 0], dib.at[0], jsems.at[0])
        pltpu.sync_copy(zeros, acc.at[pl.ds(s * RPT, RPT)])
        if with_deg:
            pltpu.sync_copy(ones_hbm, ones_v)
            pltpu.sync_copy(z16, dacc.at[pl.ds(s * RPT, RPT)])
        plsc.subcore_barrier()

        def round_body(r, carry):
            slot = lax.rem(r, 4)
            st = lax.rem(r, 2)
            oslot = lax.rem(r + 3, 4)
            ost = 1 - st

            @pl.when(r >= 2)
            def _drain():
                dslot = lax.rem(r + 2, 4)
                pltpu.make_async_copy(bufs.at[st], acc.at[dib.at[dslot]],
                                      ssems.at[st]).wait()
                if with_deg:
                    pltpu.make_async_copy(ones_v, dacc.at[dib.at[dslot]],
                                          dsems.at[st]).wait()

            @pl.when(r < R)
            def _gather():
                @pl.when(r + 1 < R)
                def _prefetch():
                    nslot = lax.rem(r + 1, 4)
                    pltpu.async_copy(src_idx.at[w, r + 1], sib.at[nslot],
                                     isems.at[nslot])
                    pltpu.async_copy(dst_idx.at[w, r + 1], dib.at[nslot],
                                     jsems.at[nslot])
                pltpu.make_async_copy(src_idx.at[w, 0], sib.at[slot],
                                      isems.at[slot]).wait()
                pltpu.async_copy(table.at[sib.at[slot]], bufs.at[st],
                                 gsems.at[st])

            @pl.when((r >= 1) & (r <= R))
            def _scatter():
                pltpu.make_async_copy(table.at[sib.at[oslot]], bufs.at[ost],
                                      gsems.at[ost]).wait()
                pltpu.make_async_copy(dst_idx.at[w, 0], dib.at[oslot],
                                      jsems.at[oslot]).wait()
                pltpu.async_copy(bufs.at[ost], acc.at[dib.at[oslot]],
                                 ssems.at[ost], add=True)
                if with_deg:
                    pltpu.async_copy(ones_v, dacc.at[dib.at[oslot]],
                                     dsems.at[ost], add=True)

            return carry

        lax.fori_loop(0, R + 2, round_body, 0)
        plsc.subcore_barrier()

        pltpu.sync_copy(acc.at[pl.ds(s * RPT, RPT)],
                        out.at[c, pl.ds(s * RPT, RPT)])
        if with_deg:
            pltpu.sync_copy(dacc.at[pl.ds(s * RPT, RPT)],
                            deg_out.at[c, pl.ds(s * RPT, RPT)])

    return pl.kernel(
        seg_sum,
        out_type=out_type if len(out_type) > 1 else out_type[0],
        mesh=mesh,
        compiler_params=pltpu.CompilerParams(use_tc_tiling_on_sc=False),
        scratch_types=scratch,
    )


_sc_seg_sum_deg = _make_sc_seg_sum(DH, with_deg=True)
_sc_seg_sum_128 = _make_sc_seg_sum(DH, with_deg=False)
_sc_seg_sum_64 = _make_sc_seg_sum(DOUT, with_deg=False)


def _row_spec(d):
    return pl.BlockSpec((BN, d), lambda i: (i, 0))


def _acc_spec(part, d):
    return pl.BlockSpec((1, BN, d), lambda i, _p=part: (_p, i, 0))


def _full_spec(r, d):
    return pl.BlockSpec((r, d), lambda i: (0, 0))


def _stage_a_body(x, wl, wr, b, t_out, hr_out):
    xb = x[...]
    t_out[...] = jnp.dot(xb, wl[...], preferred_element_type=jnp.float32)
    hr_out[...] = jnp.dot(xb, wr[...], preferred_element_type=jnp.float32) + b[...]


def _stage_a(x, wl, wr, b):
    return pl.pallas_call(
        _stage_a_body,
        grid=(GRID,),
        in_specs=[_row_spec(DIN), _full_spec(DIN, DH), _full_spec(DIN, DH),
                  _full_spec(1, DH)],
        out_specs=[_row_spec(DH), _row_spec(DH)],
        out_shape=[jax.ShapeDtypeStruct((N, DH), jnp.float32),
                   jax.ShapeDtypeStruct((N, DH), jnp.float32)],
    )(x, wl, wr, b)


def _stage_b2_body(accA, accB, degA, degB, hr, wl, wr, b,
                   t_out, hr_out, inv_out):
    a = accA[0] + accB[0]
    deg = degA[0, :, 0:1] + degB[0, :, 0:1]
    inv = 1.0 / jnp.maximum(deg, 1.0)
    h = jnp.maximum(a * inv + hr[...], 0.0)
    t_out[...] = jnp.dot(h, wl[...], preferred_element_type=jnp.float32)
    hr_out[...] = jnp.dot(h, wr[...], preferred_element_type=jnp.float32) + b[...]
    inv_out[...] = jnp.broadcast_to(inv, (BN, 8))


def _stage_b2(acc, deg, hr, wl, wr, b):
    return pl.pallas_call(
        _stage_b2_body,
        grid=(GRID,),
        in_specs=[_acc_spec(0, DH), _acc_spec(1, DH),
                  _acc_spec(0, DDEG), _acc_spec(1, DDEG), _row_spec(DH),
                  _full_spec(DH, DH), _full_spec(DH, DH), _full_spec(1, DH)],
        out_specs=[_row_spec(DH), _row_spec(DH), _row_spec(8)],
        out_shape=[jax.ShapeDtypeStruct((N, DH), jnp.float32),
                   jax.ShapeDtypeStruct((N, DH), jnp.float32),
                   jax.ShapeDtypeStruct((N, 8), jnp.float32)],
    )(acc, acc, deg, deg, hr, wl, wr, b)


def _stage_b3_body(accA, accB, hr, inv_in, wl, wr, b, t_out, hr_out):
    a = accA[0] + accB[0]
    inv = inv_in[:, 0:1]
    h = jnp.maximum(a * inv + hr[...], 0.0)
    t_out[...] = jnp.dot(h, wl[...], preferred_element_type=jnp.float32)
    hr_out[...] = jnp.dot(h, wr[...], preferred_element_type=jnp.float32) + b[...]


def _stage_b3(acc, hr, inv, wl, wr, b):
    return pl.pallas_call(
        _stage_b3_body,
        grid=(GRID,),
        in_specs=[_acc_spec(0, DH), _acc_spec(1, DH), _row_spec(DH),
                  _row_spec(8),
                  _full_spec(DH, DOUT), _full_spec(DH, DOUT), _full_spec(1, DOUT)],
        out_specs=[_row_spec(DOUT), _row_spec(DOUT)],
        out_shape=[jax.ShapeDtypeStruct((N, DOUT), jnp.float32),
                   jax.ShapeDtypeStruct((N, DOUT), jnp.float32)],
    )(acc, acc, hr, inv, wl, wr, b)


def _stage_c_body(accA, accB, hr, inv_in, out):
    a = accA[0] + accB[0]
    inv = inv_in[:, 0:1]
    h = a * inv + hr[...]
    m = jnp.max(h, axis=1, keepdims=True)
    e = jnp.exp(h - m)
    lse = jnp.log(jnp.sum(e, axis=1, keepdims=True))
    out[...] = h - m - lse


def _stage_c(acc, hr, inv):
    return pl.pallas_call(
        _stage_c_body,
        grid=(GRID,),
        in_specs=[_acc_spec(0, DOUT), _acc_spec(1, DOUT), _row_spec(DOUT),
                  _row_spec(8)],
        out_specs=_row_spec(DOUT),
        out_shape=jax.ShapeDtypeStruct((N, DOUT), jnp.float32),
    )(acc, acc, hr, inv)


def kernel(x, edge_index, W_l1, W_r1, b1, g1, be1, W_l2, W_r2, b2, g2, be2,
           W_l3, W_r3, b3):
    s1 = g1 / jnp.sqrt(1.0 + EPS)
    s2 = g2 / jnp.sqrt(1.0 + EPS)
    wl1 = W_l1 * s1
    wr1 = W_r1 * s1
    bb1 = (b1 * s1 + be1).reshape(1, DH)
    wl2 = W_l2 * s2
    wr2 = W_r2 * s2
    bb2 = (b2 * s2 + be2).reshape(1, DH)
    bb3 = b3.reshape(1, DOUT)

    src = edge_index[0].reshape(NW, T, B)
    dst = edge_index[1].reshape(NW, T, B)

    z128 = jnp.zeros((RPT, DH), jnp.float32)
    z64 = jnp.zeros((RPT, DOUT), jnp.float32)
    z16 = jnp.zeros((RPT, DDEG), jnp.float32)
    ones16 = jnp.ones((B, DDEG), jnp.float32)

    t1, hr1 = _stage_a(x, wl1, wr1, bb1)
    acc1, deg = _sc_seg_sum_deg(t1, src, dst, z128, ones16, z16)
    t2, hr2, inv = _stage_b2(acc1, deg, hr1, wl2, wr2, bb2)
    acc2 = _sc_seg_sum_128(t2, src, dst, z128)
    t3, hr3 = _stage_b3(acc2, hr2, inv, W_l3, W_r3, bb3)
    acc3 = _sc_seg_sum_64(t3, src, dst, z64)
    return _stage_c(acc3, hr3, inv)
